# baseline (speedup 1.0000x reference)
import jax
import jax.numpy as jnp
from jax import lax
from jax.experimental import pallas as pl
from jax.experimental.pallas import tpu as pltpu
from jax.experimental.pallas import tpu_sc as plsc

_B = 16384
_L = 50
_D = 64
_NW = 32
_PER_W = _B // _NW
_C = 8
_NG = _PER_W // _C
_RPG = _C * _L


def _weight_slot(l):
  if l < 32:
    return l // 16, l % 16
  if l < 34:
    return 2, l - 32
  return 3, l - 34


def _body(idx_hbm, w_hbm, table_hbm, out_hbm, idx_v, w_v, rows_v, out_v, sem):
  wid = lax.axis_index("s") * 2 + lax.axis_index("c")

  def group(g, carry):
    b0 = wid * _PER_W + g * _C
    pltpu.sync_copy(idx_hbm.at[pl.ds(b0, _C), :], idx_v)
    pltpu.sync_copy(w_hbm.at[pl.ds(b0, _C), :], w_v)
    copies = [
        pltpu.async_copy(
            table_hbm.at[idx_v.at[b]],
            rows_v.at[pl.ds(b * _L, _L)],
            sem,
        )
        for b in range(_C)
    ]
    for cp in copies:
      cp.wait()

    def bstep(b, c):
      wv = [
          w_v[b, pl.ds(0, 16)],
          w_v[b, pl.ds(16, 16)],
          w_v[b, pl.ds(32, 16)],
          w_v[b, pl.ds(34, 16)],
      ]
      accs = [jnp.zeros((16,), jnp.float32) for _ in range(4)]
      for l in range(_L):
        vr, lane = _weight_slot(l)
        ws = wv[vr][lane]
        for k in range(4):
          accs[k] = accs[k] + ws * rows_v[b * _L + l, pl.ds(k * 16, 16)]
      for k in range(4):
        out_v[b, pl.ds(k * 16, 16)] = accs[k]
      return c

    lax.fori_loop(0, _C, bstep, 0)
    pltpu.sync_copy(out_v, out_hbm.at[pl.ds(b0, _C), :])
    return carry

  lax.fori_loop(0, _NG, group, 0)


def kernel(batch_hynm, batch_hynm_weights, table):
  run = pl.kernel(
      _body,
      out_type=jax.ShapeDtypeStruct((_B, _D), jnp.float32),
      mesh=plsc.VectorSubcoreMesh(core_axis_name="c", subcore_axis_name="s"),
      scratch_types=[
          pltpu.VMEM((_C, _L), jnp.int32),
          pltpu.VMEM((_C, _L), jnp.float32),
          pltpu.VMEM((_RPG, _D), jnp.float32),
          pltpu.VMEM((_C, _D), jnp.float32),
          pltpu.SemaphoreType.DMA,
      ],
      compiler_params=pltpu.CompilerParams(use_tc_tiling_on_sc=False),
  )
  return run(batch_hynm.astype(jnp.int32), batch_hynm_weights, table)


# speedup vs baseline: 1.2939x; 1.2939x over previous
import jax
import jax.numpy as jnp
from jax import lax
from jax.experimental import pallas as pl
from jax.experimental.pallas import tpu as pltpu
from jax.experimental.pallas import tpu_sc as plsc

_B = 16384
_L = 50
_D = 64
_V = 1000000
_NW = 32
_PER_W = _B // _NW
_C = 8
_NG = _PER_W // _C
_RPG = _C * _L
_VB = 2048
_TGRID = -(-_V // (2 * _VB))
_PV = _TGRID * _VB

_SLOT_OFF = (0, 16, 32, 34)


def _weight_slot(l):
  if l < 32:
    return l // 16, l % 16
  if l < 34:
    return 2, l - 32
  return 3, l - 34


def _xpose_body(x_ref, out_ref):
  eye = (lax.broadcasted_iota(jnp.int32, (_D, _D), 0)
         == lax.broadcasted_iota(jnp.int32, (_D, _D), 1)).astype(jnp.float32)
  y = lax.dot_general(x_ref[...], eye, (((0,), (0,)), ((), ())),
                      preferred_element_type=jnp.float32)
  out_ref[:, 0:_D] = y[0:_VB, :]
  out_ref[:, _D:2 * _D] = y[_VB:2 * _VB, :]


def _pack_table(tt):
  return pl.pallas_call(
      _xpose_body,
      grid=(_TGRID,),
      in_specs=[pl.BlockSpec((_D, 2 * _VB), lambda i: (0, i))],
      out_specs=pl.BlockSpec((_VB, 2 * _D), lambda i: (i, 0)),
      out_shape=jax.ShapeDtypeStruct((_PV, 2 * _D), jnp.float32),
      compiler_params=pltpu.CompilerParams(
          dimension_semantics=("arbitrary",),
          fuse_transposed_lhs_in_matmul=True,
      ),
  )(tt)


def _sc_body(idx_hbm, w_hbm, tp_hbm, out_hbm, idx_v, idxp_v, w_v, rows_v,
             out_v, sem):
  wid = lax.axis_index("s") * 2 + lax.axis_index("c")

  def group(g, carry):
    b0 = wid * _PER_W + g * _C
    pltpu.sync_copy(idx_hbm.at[pl.ds(b0, _C), :], idx_v)
    pltpu.sync_copy(w_hbm.at[pl.ds(b0, _C), :], w_v)
    for b in range(_C):
      for off in _SLOT_OFF:
        v16 = idx_v[b, pl.ds(off, 16)]
        idxp_v[b, pl.ds(off, 16)] = (
            lax.shift_left(lax.shift_right_logical(v16, 12), 11)
            + (v16 & 2047))
    copies = [
        pltpu.async_copy(
            tp_hbm.at[idxp_v.at[b]],
            rows_v.at[pl.ds(b * _L, _L)],
            sem,
        )
        for b in range(_C)
    ]
    for cp in copies:
      cp.wait()

    def bstep(b, c):
      wv = [w_v[b, pl.ds(off, 16)] for off in _SLOT_OFF]
      iv = [idx_v[b, pl.ds(off, 16)] for off in _SLOT_OFF]
      accs = [jnp.zeros((16,), jnp.float32) for _ in range(4)]
      for l in range(_L):
        vr, lane = _weight_slot(l)
        ws = wv[vr][lane]
        half = lax.shift_right_logical(iv[vr][lane], 5) & _D
        for k in range(4):
          accs[k] = accs[k] + ws * rows_v[b * _L + l,
                                          pl.ds(half + k * 16, 16)]
      for k in range(4):
        out_v[b, pl.ds(k * 16, 16)] = accs[k]
      return c

    lax.fori_loop(0, _C, bstep, 0)
    pltpu.sync_copy(out_v, out_hbm.at[pl.ds(b0, _C), :])
    return carry

  lax.fori_loop(0, _NG, group, 0)


def kernel(batch_hynm, batch_hynm_weights, table):
  tp = _pack_table(table.T)
  run = pl.kernel(
      _sc_body,
      out_type=jax.ShapeDtypeStruct((_B, _D), jnp.float32),
      mesh=plsc.VectorSubcoreMesh(core_axis_name="c", subcore_axis_name="s"),
      scratch_types=[
          pltpu.VMEM((_C, _L), jnp.int32),
          pltpu.VMEM((_C, _L), jnp.int32),
          pltpu.VMEM((_C, _L), jnp.float32),
          pltpu.VMEM((_RPG, 2 * _D), jnp.float32),
          pltpu.VMEM((_C, _D), jnp.float32),
          pltpu.SemaphoreType.DMA,
      ],
  )
  return run(batch_hynm.astype(jnp.int32), batch_hynm_weights, tp)


# speedup vs baseline: 1.6162x; 1.2491x over previous
import jax
import jax.numpy as jnp
from jax import lax
from jax.experimental import pallas as pl
from jax.experimental.pallas import tpu as pltpu
from jax.experimental.pallas import tpu_sc as plsc

_B = 16384
_L = 50
_D = 64
_V = 1000000
_NW = 32
_PER_W = _B // _NW
_C = 8
_NG = _PER_W // _C
_RPG = _C * _L
_VB = 2048
_TGRID = -(-_V // (2 * _VB))
_PV = _TGRID * _VB

_SLOT_OFF = (0, 16, 32, 34)


def _weight_slot(l):
  if l < 32:
    return l // 16, l % 16
  if l < 34:
    return 2, l - 32
  return 3, l - 34


def _xpose_body(x_ref, out_ref):
  eye = (lax.broadcasted_iota(jnp.int32, (_D, _D), 0)
         == lax.broadcasted_iota(jnp.int32, (_D, _D), 1)).astype(jnp.float32)
  y = lax.dot_general(x_ref[...], eye, (((0,), (0,)), ((), ())),
                      preferred_element_type=jnp.float32)
  out_ref[:, 0:_D] = y[0:_VB, :]
  out_ref[:, _D:2 * _D] = y[_VB:2 * _VB, :]


def _pack_table(tt):
  return pl.pallas_call(
      _xpose_body,
      grid=(_TGRID,),
      in_specs=[pl.BlockSpec((_D, 2 * _VB), lambda i: (0, i))],
      out_specs=pl.BlockSpec((_VB, 2 * _D), lambda i: (i, 0)),
      out_shape=jax.ShapeDtypeStruct((_PV, 2 * _D), jnp.float32),
      compiler_params=pltpu.CompilerParams(
          dimension_semantics=("arbitrary",),
          fuse_transposed_lhs_in_matmul=True,
      ),
  )(tt)


def _stage(g, wid, idx_hbm, w_hbm, idx_v, idxp_v, w_v):
  b0 = wid * _PER_W + g * _C
  pltpu.sync_copy(idx_hbm.at[pl.ds(b0, _C), :], idx_v)
  pltpu.sync_copy(w_hbm.at[pl.ds(b0, _C), :], w_v)
  for b in range(_C):
    for off in _SLOT_OFF:
      v16 = idx_v[b, pl.ds(off, 16)]
      idxp_v[b, pl.ds(off, 16)] = (
          lax.shift_left(lax.shift_right_logical(v16, 12), 11)
          + (v16 & 2047))


def _fire(tp_hbm, idxp_v, rows_v, sem):
  for b in range(_C):
    pltpu.async_copy(
        tp_hbm.at[idxp_v.at[b]],
        rows_v.at[pl.ds(b * _L, _L)],
        sem,
    )


def _wait(tp_hbm, rows_v, sem):
  pltpu.make_async_copy(tp_hbm.at[pl.ds(0, _RPG), :], rows_v, sem).wait()


def _compute(g, wid, idx_v, w_v, rows_v, out_v, out_hbm):
  b0 = wid * _PER_W + g * _C

  def bstep(b, c):
    wv = [w_v[b, pl.ds(off, 16)] for off in _SLOT_OFF]
    iv = [idx_v[b, pl.ds(off, 16)] for off in _SLOT_OFF]
    accs = [jnp.zeros((16,), jnp.float32) for _ in range(4)]
    for l in range(_L):
      vr, lane = _weight_slot(l)
      ws = wv[vr][lane]
      half = lax.shift_right_logical(iv[vr][lane], 5) & _D
      for k in range(4):
        accs[k] = accs[k] + ws * rows_v[b * _L + l,
                                        pl.ds(half + k * 16, 16)]
    for k in range(4):
      out_v[b, pl.ds(k * 16, 16)] = accs[k]
    return c

  lax.fori_loop(0, _C, bstep, 0)
  pltpu.sync_copy(out_v, out_hbm.at[pl.ds(b0, _C), :])


def _sc_body(idx_hbm, w_hbm, tp_hbm, out_hbm,
             idx_v0, idxp_v0, w_v0, rows_v0,
             idx_v1, idxp_v1, w_v1, rows_v1,
             out_v, sem0, sem1):
  wid = lax.axis_index("s") * 2 + lax.axis_index("c")

  _stage(0, wid, idx_hbm, w_hbm, idx_v0, idxp_v0, w_v0)
  _fire(tp_hbm, idxp_v0, rows_v0, sem0)
  _stage(1, wid, idx_hbm, w_hbm, idx_v1, idxp_v1, w_v1)
  _fire(tp_hbm, idxp_v1, rows_v1, sem1)

  def pair(h, carry):
    g0 = 2 * h
    _wait(tp_hbm, rows_v0, sem0)
    _compute(g0, wid, idx_v0, w_v0, rows_v0, out_v, out_hbm)

    @pl.when(h < _NG // 2 - 1)
    def _():
      _stage(g0 + 2, wid, idx_hbm, w_hbm, idx_v0, idxp_v0, w_v0)
      _fire(tp_hbm, idxp_v0, rows_v0, sem0)

    _wait(tp_hbm, rows_v1, sem1)
    _compute(g0 + 1, wid, idx_v1, w_v1, rows_v1, out_v, out_hbm)

    @pl.when(h < _NG // 2 - 1)
    def _():
      _stage(g0 + 3, wid, idx_hbm, w_hbm, idx_v1, idxp_v1, w_v1)
      _fire(tp_hbm, idxp_v1, rows_v1, sem1)

    return carry

  lax.fori_loop(0, _NG // 2, pair, 0)


def kernel(batch_hynm, batch_hynm_weights, table):
  tp = _pack_table(table.T)
  run = pl.kernel(
      _sc_body,
      out_type=jax.ShapeDtypeStruct((_B, _D), jnp.float32),
      mesh=plsc.VectorSubcoreMesh(core_axis_name="c", subcore_axis_name="s"),
      scratch_types=[
          pltpu.VMEM((_C, _L), jnp.int32),
          pltpu.VMEM((_C, _L), jnp.int32),
          pltpu.VMEM((_C, _L), jnp.float32),
          pltpu.VMEM((_RPG, 2 * _D), jnp.float32),
          pltpu.VMEM((_C, _L), jnp.int32),
          pltpu.VMEM((_C, _L), jnp.int32),
          pltpu.VMEM((_C, _L), jnp.float32),
          pltpu.VMEM((_RPG, 2 * _D), jnp.float32),
          pltpu.VMEM((_C, _D), jnp.float32),
          pltpu.SemaphoreType.DMA,
          pltpu.SemaphoreType.DMA,
      ],
  )
  return run(batch_hynm.astype(jnp.int32), batch_hynm_weights, tp)


# speedup vs baseline: 2.2802x; 1.4109x over previous
import jax
import jax.numpy as jnp
from jax import lax
from jax.experimental import pallas as pl
from jax.experimental.pallas import tpu as pltpu
from jax.experimental.pallas import tpu_sc as plsc

_B = 16384
_L = 50
_D = 64
_V = 1000000
_NW = 32
_PER_W = _B // _NW
_C = 8
_NG = _PER_W // _C
_RPG = _C * _L
_VB = 8192
_VBITS = 14
_TGRID = -(-_V // (2 * _VB))
_PV = _TGRID * _VB

_SLOT_OFF = (0, 16, 32, 34)


def _weight_slot(l):
  if l < 32:
    return l // 16, l % 16
  if l < 34:
    return 2, l - 32
  return 3, l - 34


def _xpose_body(x_ref, out_ref):
  eye = (lax.broadcasted_iota(jnp.int32, (2 * _D, 2 * _D), 0)
         == lax.broadcasted_iota(jnp.int32, (2 * _D, 2 * _D), 1)
         ).astype(jnp.float32)
  x = x_ref[...]
  xstack = jnp.concatenate([x[:, 0:_VB], x[:, _VB:2 * _VB]], axis=0)
  out_ref[...] = lax.dot_general(xstack, eye, (((0,), (0,)), ((), ())),
                                 preferred_element_type=jnp.float32)


def _pack_table(tt):
  return pl.pallas_call(
      _xpose_body,
      grid=(_TGRID,),
      in_specs=[pl.BlockSpec((_D, 2 * _VB), lambda i: (0, i))],
      out_specs=pl.BlockSpec((_VB, 2 * _D), lambda i: (i, 0)),
      out_shape=jax.ShapeDtypeStruct((_PV, 2 * _D), jnp.float32),
      compiler_params=pltpu.CompilerParams(
          dimension_semantics=("arbitrary",),
          fuse_transposed_lhs_in_matmul=True,
      ),
  )(tt)


def _stage(g, wid, idx_hbm, w_hbm, idx_v, idxp_v, w_v):
  b0 = wid * _PER_W + g * _C
  pltpu.sync_copy(idx_hbm.at[pl.ds(b0, _C), :], idx_v)
  pltpu.sync_copy(w_hbm.at[pl.ds(b0, _C), :], w_v)
  for b in range(_C):
    for off in _SLOT_OFF:
      v16 = idx_v[b, pl.ds(off, 16)]
      idxp_v[b, pl.ds(off, 16)] = (
          lax.shift_left(lax.shift_right_logical(v16, _VBITS), _VBITS - 1)
          + (v16 & (_VB - 1)))


def _fire(tp_hbm, idxp_v, rows_v, sem):
  for b in range(_C):
    pltpu.async_copy(
        tp_hbm.at[idxp_v.at[b]],
        rows_v.at[pl.ds(b * _L, _L)],
        sem,
    )


def _wait(tp_hbm, rows_v, sem):
  pltpu.make_async_copy(tp_hbm.at[pl.ds(0, _RPG), :], rows_v, sem).wait()


def _compute(g, wid, idx_v, w_v, rows_v, out_v, out_hbm):
  b0 = wid * _PER_W + g * _C

  def bstep(b, c):
    wv = [w_v[b, pl.ds(off, 16)] for off in _SLOT_OFF]
    iv = [idx_v[b, pl.ds(off, 16)] for off in _SLOT_OFF]
    accs = [jnp.zeros((16,), jnp.float32) for _ in range(4)]
    for l in range(_L):
      vr, lane = _weight_slot(l)
      ws = wv[vr][lane]
      half = lax.shift_right_logical(iv[vr][lane], _VBITS - 7) & _D
      for k in range(4):
        accs[k] = accs[k] + ws * rows_v[b * _L + l,
                                        pl.ds(half + k * 16, 16)]
    for k in range(4):
      out_v[b, pl.ds(k * 16, 16)] = accs[k]
    return c

  lax.fori_loop(0, _C, bstep, 0)
  pltpu.sync_copy(out_v, out_hbm.at[pl.ds(b0, _C), :])


def _sc_body(idx_hbm, w_hbm, tp_hbm, out_hbm,
             idx_v0, idxp_v0, w_v0, rows_v0,
             idx_v1, idxp_v1, w_v1, rows_v1,
             out_v, sem0, sem1):
  wid = lax.axis_index("s") * 2 + lax.axis_index("c")

  _stage(0, wid, idx_hbm, w_hbm, idx_v0, idxp_v0, w_v0)
  _fire(tp_hbm, idxp_v0, rows_v0, sem0)
  _stage(1, wid, idx_hbm, w_hbm, idx_v1, idxp_v1, w_v1)
  _fire(tp_hbm, idxp_v1, rows_v1, sem1)

  def pair(h, carry):
    g0 = 2 * h
    _wait(tp_hbm, rows_v0, sem0)
    _compute(g0, wid, idx_v0, w_v0, rows_v0, out_v, out_hbm)

    @pl.when(h < _NG // 2 - 1)
    def _():
      _stage(g0 + 2, wid, idx_hbm, w_hbm, idx_v0, idxp_v0, w_v0)
      _fire(tp_hbm, idxp_v0, rows_v0, sem0)

    _wait(tp_hbm, rows_v1, sem1)
    _compute(g0 + 1, wid, idx_v1, w_v1, rows_v1, out_v, out_hbm)

    @pl.when(h < _NG // 2 - 1)
    def _():
      _stage(g0 + 3, wid, idx_hbm, w_hbm, idx_v1, idxp_v1, w_v1)
      _fire(tp_hbm, idxp_v1, rows_v1, sem1)

    return carry

  lax.fori_loop(0, _NG // 2, pair, 0)


def kernel(batch_hynm, batch_hynm_weights, table):
  tp = _pack_table(table.T)
  run = pl.kernel(
      _sc_body,
      out_type=jax.ShapeDtypeStruct((_B, _D), jnp.float32),
      mesh=plsc.VectorSubcoreMesh(core_axis_name="c", subcore_axis_name="s"),
      scratch_types=[
          pltpu.VMEM((_C, _L), jnp.int32),
          pltpu.VMEM((_C, _L), jnp.int32),
          pltpu.VMEM((_C, _L), jnp.float32),
          pltpu.VMEM((_RPG, 2 * _D), jnp.float32),
          pltpu.VMEM((_C, _L), jnp.int32),
          pltpu.VMEM((_C, _L), jnp.int32),
          pltpu.VMEM((_C, _L), jnp.float32),
          pltpu.VMEM((_RPG, 2 * _D), jnp.float32),
          pltpu.VMEM((_C, _D), jnp.float32),
          pltpu.SemaphoreType.DMA,
          pltpu.SemaphoreType.DMA,
      ],
  )
  return run(batch_hynm.astype(jnp.int32), batch_hynm_weights, tp)
